# baseline (speedup 1.0000x reference)
import jax
import jax.numpy as jnp
from jax import lax
from jax.experimental import pallas as pl
from jax.experimental.pallas import tpu as pltpu
from jax.experimental.pallas import tpu_sc as plsc

F = 26
B = 4096
L = 20
D = 32
LANES = 16

NC, NS = 2, 16
NW = NC * NS
BPW = B // NW

CB = 32
NSUB = BPW // CB
NCH = F * NSUB
LP = 24
IDS = CB * LP


def _sc_embedding_bag(ids_hbm, table_hbm, out_hbm,
                      ids0, ids1, rows0, rows1, out0, out1,
                      semg0, semg1, semi, semo0, semo1):
    wid = lax.axis_index("s") * NC + lax.axis_index("c")
    b0 = wid * BPW

    def ids_off(t):
        f = t // NSUB
        bb = b0 + (t % NSUB) * CB
        return bb, f

    def stage_ids(t, idsb):
        bb, f = ids_off(t)
        pltpu.async_copy(
            ids_hbm.at[f, pl.ds(bb, CB), pl.ds(0, LP)], idsb, semi)

    def wait_ids(idsb):
        pltpu.make_async_copy(
            ids_hbm.at[0, pl.ds(0, CB), pl.ds(0, LP)], idsb, semi).wait()

    def fire_gathers(idsb, rowsb, semg):
        for i in range(CB):
            pltpu.async_copy(
                table_hbm.at[idsb.at[i]],
                rowsb.at[pl.ds(i * LP, LP)],
                semg,
            )

    def drain_gathers(rowsb, semg):
        pltpu.make_async_copy(
            table_hbm.at[pl.ds(0, IDS)], rowsb, semg).wait()

    def compute(rowsb, outb):
        def bag_body(bag, c):
            base = bag * LP
            acc0 = rowsb[base, pl.ds(0, LANES)]
            acc1 = rowsb[base, pl.ds(LANES, LANES)]
            for l in range(1, L):
                acc0 = acc0 + rowsb[base + l, pl.ds(0, LANES)]
                acc1 = acc1 + rowsb[base + l, pl.ds(LANES, LANES)]
            outb[bag, pl.ds(0, LANES)] = acc0
            outb[bag, pl.ds(LANES, LANES)] = acc1
            return c

        lax.fori_loop(0, CB, bag_body, 0, unroll=False)

    def fire_out(t, outb, semo):
        bb, f = ids_off(t)
        pltpu.async_copy(outb, out_hbm.at[pl.ds(bb, CB), f], semo)

    def drain_out(outb, semo):
        pltpu.make_async_copy(outb, out_hbm.at[pl.ds(0, CB), 0], semo).wait()

    pltpu.sync_copy(ids_hbm.at[0, pl.ds(b0, CB), pl.ds(0, LP)], ids0)
    fire_gathers(ids0, rows0, semg0)
    stage_ids(1, ids1)

    def half(t, idsb_n, rowsb_n, semg_n, idsb_c, rowsb_c, semg_c,
             outb, semo, skip_out_drain):
        wait_ids(idsb_n)
        fire_gathers(idsb_n, rowsb_n, semg_n)
        drain_gathers(rowsb_c, semg_c)
        stage_ids(t + 2, idsb_c)
        if not skip_out_drain:
            drain_out(outb, semo)
        compute(rowsb_c, outb)
        fire_out(t, outb, semo)

    def make_pair_body(first_pair):
        def pair_body(k, carry):
            t0 = 2 * k
            half(t0, ids1, rows1, semg1, ids0, rows0, semg0,
                 out0, semo0, first_pair)
            half(t0 + 1, ids0, rows0, semg0, ids1, rows1, semg1,
                 out1, semo1, first_pair)
            return carry
        return pair_body

    make_pair_body(True)(0, 0)
    lax.fori_loop(1, NCH // 2 - 1, make_pair_body(False), 0, unroll=False)

    t = NCH - 2
    wait_ids(ids1)
    fire_gathers(ids1, rows1, semg1)
    drain_gathers(rows0, semg0)
    drain_out(out0, semo0)
    compute(rows0, out0)
    fire_out(t, out0, semo0)

    drain_gathers(rows1, semg1)
    drain_out(out1, semo1)
    compute(rows1, out1)
    fire_out(t + 1, out1, semo1)

    drain_out(out0, semo0)
    drain_out(out1, semo1)


@jax.jit
def _run(indices, table):
    kern = pl.kernel(
        _sc_embedding_bag,
        out_type=jax.ShapeDtypeStruct((B, F, D), jnp.float32),
        mesh=plsc.VectorSubcoreMesh(core_axis_name="c", subcore_axis_name="s"),
        scratch_types=[
            pltpu.VMEM((CB, LP), jnp.int32),
            pltpu.VMEM((CB, LP), jnp.int32),
            pltpu.VMEM((IDS, D), jnp.float32),
            pltpu.VMEM((IDS, D), jnp.float32),
            pltpu.VMEM((CB, D), jnp.float32),
            pltpu.VMEM((CB, D), jnp.float32),
            pltpu.SemaphoreType.DMA,
            pltpu.SemaphoreType.DMA,
            pltpu.SemaphoreType.DMA,
            pltpu.SemaphoreType.DMA,
            pltpu.SemaphoreType.DMA,
        ],
        compiler_params=pltpu.CompilerParams(use_tc_tiling_on_sc=False),
    )
    idx_pad = jnp.pad(indices, ((0, 0), (0, 0), (0, 128 - L)))
    out = kern(idx_pad, table)
    return out.reshape(B, F * D)


def kernel(indices, table):
    return _run(indices.astype(jnp.int32), table.astype(jnp.float32))


# speedup vs baseline: 6.9722x; 6.9722x over previous
import jax
import jax.numpy as jnp
from jax import lax
from jax.experimental import pallas as pl
from jax.experimental.pallas import tpu as pltpu
from jax.experimental.pallas import tpu_sc as plsc

F = 26
B = 4096
L = 20
D = 32
LANES = 16

NC, NS = 2, 16
NW = NC * NS
BPW = B // NW

CB = 64
NSUB = BPW // CB
NCH = F * NSUB
IDS = CB * L
GW = 128
NG = IDS // GW


def _sc_depad_ids(ids3_hbm, flat_hbm, in0, in1, fl0, fl1,
                  semi, semo0, semo1):
    wid = lax.axis_index("s") * NC + lax.axis_index("c")
    b0 = wid * BPW

    def stage(f, inb):
        pltpu.async_copy(ids3_hbm.at[f, pl.ds(b0, BPW), :], inb, semi)

    def wait_in(inb):
        pltpu.make_async_copy(
            ids3_hbm.at[0, pl.ds(0, BPW), :], inb, semi).wait()

    def flatten(inb, flb):
        def row_body(i, c):
            flb[pl.ds(i * L, LANES)] = inb[i, pl.ds(0, LANES)]
            flb[pl.ds(i * L + L - LANES, LANES)] = inb[i, pl.ds(L - LANES,
                                                                LANES)]
            return c
        lax.fori_loop(0, BPW, row_body, 0, unroll=False)

    def fire_out(f, flb, semo):
        o = f * (B * L) + b0 * L
        pltpu.async_copy(flb, flat_hbm.at[pl.ds(o, BPW * L)], semo)

    def drain_out(flb, semo):
        pltpu.make_async_copy(
            flb, flat_hbm.at[pl.ds(0, BPW * L)], semo).wait()

    def step(f, inb_c, flb, semo, inb_n, do_stage, do_drain):
        wait_in(inb_c)
        if do_stage:
            stage(f + 1, inb_n)
        if do_drain:
            drain_out(flb, semo)
        flatten(inb_c, flb)
        fire_out(f, flb, semo)

    stage(0, in0)
    step(0, in0, fl0, semo0, in1, True, False)
    step(1, in1, fl1, semo1, in0, True, False)

    def pair_body(k, carry):
        f0 = 2 * k
        step(f0, in0, fl0, semo0, in1, True, True)
        step(f0 + 1, in1, fl1, semo1, in0, True, True)
        return carry

    lax.fori_loop(1, F // 2 - 1, pair_body, 0, unroll=False)

    step(F - 2, in0, fl0, semo0, in1, True, True)
    step(F - 1, in1, fl1, semo1, in0, False, True)
    drain_out(fl0, semo0)
    drain_out(fl1, semo1)


def _sc_embedding_bag(ids_hbm, table_hbm, out_hbm,
                      ids0, ids1, rows0, rows1, out0, out1,
                      semg0, semg1, semi, semo0, semo1):
    wid = lax.axis_index("s") * NC + lax.axis_index("c")
    b0 = wid * BPW

    def ids_off(t):
        f = t // NSUB
        bb = b0 + (t % NSUB) * CB
        return f * (B * L) + bb * L, bb, f

    def stage_ids(t, idsb):
        o0, _, _ = ids_off(t)
        pltpu.async_copy(ids_hbm.at[pl.ds(o0, IDS)], idsb, semi)

    def wait_ids(idsb):
        pltpu.make_async_copy(ids_hbm.at[pl.ds(0, IDS)], idsb, semi).wait()

    def fire_gathers(idsb, rowsb, semg):
        for j in range(NG):
            pltpu.async_copy(
                table_hbm.at[idsb.at[pl.ds(j * GW, GW)]],
                rowsb.at[pl.ds(j * GW, GW)],
                semg,
            )

    def drain_gathers(rowsb, semg):
        pltpu.make_async_copy(
            table_hbm.at[pl.ds(0, IDS)], rowsb, semg).wait()

    def compute(rowsb, outb):
        def bag_body(bag, c):
            base = bag * L
            acc0 = rowsb[base, pl.ds(0, LANES)]
            acc1 = rowsb[base, pl.ds(LANES, LANES)]
            for l in range(1, L):
                acc0 = acc0 + rowsb[base + l, pl.ds(0, LANES)]
                acc1 = acc1 + rowsb[base + l, pl.ds(LANES, LANES)]
            outb[bag, pl.ds(0, LANES)] = acc0
            outb[bag, pl.ds(LANES, LANES)] = acc1
            return c

        lax.fori_loop(0, CB, bag_body, 0, unroll=False)

    def fire_out(t, outb, semo):
        _, bb, f = ids_off(t)
        pltpu.async_copy(outb, out_hbm.at[pl.ds(bb, CB), f], semo)

    def drain_out(outb, semo):
        pltpu.make_async_copy(outb, out_hbm.at[pl.ds(0, CB), 0], semo).wait()

    pltpu.sync_copy(ids_hbm.at[pl.ds(b0 * L, IDS)], ids0)
    fire_gathers(ids0, rows0, semg0)
    stage_ids(1, ids1)

    def half(t, idsb_n, rowsb_n, semg_n, idsb_c, rowsb_c, semg_c,
             outb, semo, skip_out_drain):
        wait_ids(idsb_n)
        fire_gathers(idsb_n, rowsb_n, semg_n)
        drain_gathers(rowsb_c, semg_c)
        stage_ids(t + 2, idsb_c)
        if not skip_out_drain:
            drain_out(outb, semo)
        compute(rowsb_c, outb)
        fire_out(t, outb, semo)

    def make_pair_body(first_pair):
        def pair_body(k, carry):
            t0 = 2 * k
            half(t0, ids1, rows1, semg1, ids0, rows0, semg0,
                 out0, semo0, first_pair)
            half(t0 + 1, ids0, rows0, semg0, ids1, rows1, semg1,
                 out1, semo1, first_pair)
            return carry
        return pair_body

    make_pair_body(True)(0, 0)
    lax.fori_loop(1, NCH // 2 - 1, make_pair_body(False), 0, unroll=False)

    t = NCH - 2
    wait_ids(ids1)
    fire_gathers(ids1, rows1, semg1)
    drain_gathers(rows0, semg0)
    drain_out(out0, semo0)
    compute(rows0, out0)
    fire_out(t, out0, semo0)

    drain_gathers(rows1, semg1)
    drain_out(out1, semo1)
    compute(rows1, out1)
    fire_out(t + 1, out1, semo1)

    drain_out(out0, semo0)
    drain_out(out1, semo1)


@jax.jit
def _run(indices, table):
    depad = pl.kernel(
        _sc_depad_ids,
        out_type=jax.ShapeDtypeStruct((F * B * L,), jnp.int32),
        mesh=plsc.VectorSubcoreMesh(core_axis_name="c", subcore_axis_name="s"),
        scratch_types=[
            pltpu.VMEM((BPW, L), jnp.int32),
            pltpu.VMEM((BPW, L), jnp.int32),
            pltpu.VMEM((BPW * L,), jnp.int32),
            pltpu.VMEM((BPW * L,), jnp.int32),
            pltpu.SemaphoreType.DMA,
            pltpu.SemaphoreType.DMA,
            pltpu.SemaphoreType.DMA,
        ],
    )
    ids_hbm = depad(indices)
    kern = pl.kernel(
        _sc_embedding_bag,
        out_type=jax.ShapeDtypeStruct((B, F, D), jnp.float32),
        mesh=plsc.VectorSubcoreMesh(core_axis_name="c", subcore_axis_name="s"),
        scratch_types=[
            pltpu.VMEM((IDS,), jnp.int32),
            pltpu.VMEM((IDS,), jnp.int32),
            pltpu.VMEM((IDS, D), jnp.float32),
            pltpu.VMEM((IDS, D), jnp.float32),
            pltpu.VMEM((CB, D), jnp.float32),
            pltpu.VMEM((CB, D), jnp.float32),
            pltpu.SemaphoreType.DMA,
            pltpu.SemaphoreType.DMA,
            pltpu.SemaphoreType.DMA,
            pltpu.SemaphoreType.DMA,
            pltpu.SemaphoreType.DMA,
        ],
        compiler_params=pltpu.CompilerParams(use_tc_tiling_on_sc=False),
    )
    out = kern(ids_hbm, table)
    return out.reshape(B, F * D)


def kernel(indices, table):
    return _run(indices.astype(jnp.int32), table.astype(jnp.float32))


# speedup vs baseline: 7.3172x; 1.0495x over previous
import jax
import jax.numpy as jnp
from jax import lax
from jax.experimental import pallas as pl
from jax.experimental.pallas import tpu as pltpu
from jax.experimental.pallas import tpu_sc as plsc

F = 26
B = 4096
L = 20
D = 32
LANES = 16

NUM_EMB = 1000000

NC, NS = 2, 16
NW = NC * NS
BPW = B // NW

CB = 64
NSUB = BPW // CB
NCH = F * NSUB
IDS = CB * L
GW = 128
NG = IDS // GW


def _sc_depad_ids(ids3_hbm, flat_hbm, in0, in1, fl0, fl1,
                  semi, semo0, semo1):
    wid = lax.axis_index("s") * NC + lax.axis_index("c")
    b0 = wid * BPW

    def stage(f, inb):
        pltpu.async_copy(ids3_hbm.at[f, pl.ds(b0, BPW), :], inb, semi)

    def wait_in(inb):
        pltpu.make_async_copy(
            ids3_hbm.at[0, pl.ds(0, BPW), :], inb, semi).wait()

    def flatten(inb, flb):
        def row_body(i, c):
            flb[pl.ds(i * L, LANES)] = inb[i, pl.ds(0, LANES)]
            flb[pl.ds(i * L + L - LANES, LANES)] = inb[i, pl.ds(L - LANES,
                                                                LANES)]
            return c
        lax.fori_loop(0, BPW, row_body, 0, unroll=False)

    def fire_out(f, flb, semo):
        o = f * (B * L) + b0 * L
        pltpu.async_copy(flb, flat_hbm.at[pl.ds(o, BPW * L)], semo)

    def drain_out(flb, semo):
        pltpu.make_async_copy(
            flb, flat_hbm.at[pl.ds(0, BPW * L)], semo).wait()

    def step(f, inb_c, flb, semo, inb_n, do_stage, do_drain):
        wait_in(inb_c)
        if do_stage:
            stage(f + 1, inb_n)
        if do_drain:
            drain_out(flb, semo)
        flatten(inb_c, flb)
        fire_out(f, flb, semo)

    stage(0, in0)
    step(0, in0, fl0, semo0, in1, True, False)
    step(1, in1, fl1, semo1, in0, True, False)

    def pair_body(k, carry):
        f0 = 2 * k
        step(f0, in0, fl0, semo0, in1, True, True)
        step(f0 + 1, in1, fl1, semo1, in0, True, True)
        return carry

    lax.fori_loop(1, F // 2 - 1, pair_body, 0, unroll=False)

    step(F - 2, in0, fl0, semo0, in1, True, True)
    step(F - 1, in1, fl1, semo1, in0, False, True)
    drain_out(fl0, semo0)
    drain_out(fl1, semo1)


def _sc_embedding_bag(ids_hbm, table_hbm, out_hbm,
                      ids0, ids1, rows0, rows1, out0, out1,
                      semg0, semg1, semi, semo0, semo1):
    wid = lax.axis_index("s") * NC + lax.axis_index("c")
    b0 = wid * BPW

    def ids_off(t):
        f = t // NSUB
        bb = b0 + (t % NSUB) * CB
        return f * (B * L) + bb * L, bb, f

    def stage_ids(t, idsb):
        o0, _, _ = ids_off(t)
        pltpu.async_copy(ids_hbm.at[pl.ds(o0, IDS)], idsb, semi)

    def wait_ids(idsb):
        pltpu.make_async_copy(ids_hbm.at[pl.ds(0, IDS)], idsb, semi).wait()

    def fire_gathers(idsb, rowsb, semg):
        for j in range(NG):
            pltpu.async_copy(
                table_hbm.at[idsb.at[pl.ds(j * GW, GW)]],
                rowsb.at[pl.ds(j * GW, GW)],
                semg,
            )

    def drain_gathers(rowsb, semg):
        pltpu.make_async_copy(
            table_hbm.at[pl.ds(0, IDS)], rowsb, semg).wait()

    def compute(rowsb, outb):
        def bag_body(bag, c):
            base = bag * L
            acc0 = rowsb[base, pl.ds(0, LANES)]
            acc1 = rowsb[base, pl.ds(LANES, LANES)]
            for l in range(1, L):
                acc0 = acc0 + rowsb[base + l, pl.ds(0, LANES)]
                acc1 = acc1 + rowsb[base + l, pl.ds(LANES, LANES)]
            outb[bag, pl.ds(0, LANES)] = acc0
            outb[bag, pl.ds(LANES, LANES)] = acc1
            return c

        lax.fori_loop(0, CB, bag_body, 0, unroll=False)

    def fire_out(t, outb, semo):
        _, bb, f = ids_off(t)
        pltpu.async_copy(
            outb, out_hbm.at[pl.ds(bb, CB), pl.ds(f * D, D)], semo)

    def drain_out(outb, semo):
        pltpu.make_async_copy(
            outb, out_hbm.at[pl.ds(0, CB), pl.ds(0, D)], semo).wait()

    pltpu.sync_copy(ids_hbm.at[pl.ds(b0 * L, IDS)], ids0)
    fire_gathers(ids0, rows0, semg0)
    stage_ids(1, ids1)

    def half(t, idsb_n, rowsb_n, semg_n, idsb_c, rowsb_c, semg_c,
             outb, semo, skip_out_drain):
        wait_ids(idsb_n)
        fire_gathers(idsb_n, rowsb_n, semg_n)
        drain_gathers(rowsb_c, semg_c)
        stage_ids(t + 2, idsb_c)
        if not skip_out_drain:
            drain_out(outb, semo)
        compute(rowsb_c, outb)
        fire_out(t, outb, semo)

    def make_pair_body(first_pair):
        def pair_body(k, carry):
            t0 = 2 * k
            half(t0, ids1, rows1, semg1, ids0, rows0, semg0,
                 out0, semo0, first_pair)
            half(t0 + 1, ids0, rows0, semg0, ids1, rows1, semg1,
                 out1, semo1, first_pair)
            return carry
        return pair_body

    make_pair_body(True)(0, 0)
    lax.fori_loop(1, NCH // 2 - 1, make_pair_body(False), 0, unroll=False)

    t = NCH - 2
    wait_ids(ids1)
    fire_gathers(ids1, rows1, semg1)
    drain_gathers(rows0, semg0)
    drain_out(out0, semo0)
    compute(rows0, out0)
    fire_out(t, out0, semo0)

    drain_gathers(rows1, semg1)
    drain_out(out1, semo1)
    compute(rows1, out1)
    fire_out(t + 1, out1, semo1)

    drain_out(out0, semo0)
    drain_out(out1, semo1)


@jax.jit
def _run(indices, table):
    depad = pl.kernel(
        _sc_depad_ids,
        out_type=jax.ShapeDtypeStruct((F * B * L,), jnp.int32),
        mesh=plsc.VectorSubcoreMesh(core_axis_name="c", subcore_axis_name="s"),
        scratch_types=[
            pltpu.VMEM((BPW, L), jnp.int32),
            pltpu.VMEM((BPW, L), jnp.int32),
            pltpu.VMEM((BPW * L,), jnp.int32),
            pltpu.VMEM((BPW * L,), jnp.int32),
            pltpu.SemaphoreType.DMA,
            pltpu.SemaphoreType.DMA,
            pltpu.SemaphoreType.DMA,
        ],
    )
    ids_hbm = depad(indices)
    kern = pl.kernel(
        _sc_embedding_bag,
        out_type=jax.ShapeDtypeStruct((B, F * D), jnp.float32),
        mesh=plsc.VectorSubcoreMesh(core_axis_name="c", subcore_axis_name="s"),
        scratch_types=[
            pltpu.VMEM((IDS,), jnp.int32),
            pltpu.VMEM((IDS,), jnp.int32),
            pltpu.VMEM((IDS, D), jnp.float32),
            pltpu.VMEM((IDS, D), jnp.float32),
            pltpu.VMEM((CB, D), jnp.float32),
            pltpu.VMEM((CB, D), jnp.float32),
            pltpu.SemaphoreType.DMA,
            pltpu.SemaphoreType.DMA,
            pltpu.SemaphoreType.DMA,
            pltpu.SemaphoreType.DMA,
            pltpu.SemaphoreType.DMA,
        ],
        compiler_params=pltpu.CompilerParams(use_tc_tiling_on_sc=False),
    )
    return kern(ids_hbm, table)


def kernel(indices, table):
    return _run(indices.astype(jnp.int32), table.astype(jnp.float32))


# speedup vs baseline: 7.3197x; 1.0003x over previous
import jax
import jax.numpy as jnp
from jax import lax
from jax.experimental import pallas as pl
from jax.experimental.pallas import tpu as pltpu
from jax.experimental.pallas import tpu_sc as plsc

F = 26
B = 4096
L = 20
D = 32
LANES = 16

NC, NS = 2, 16
NW = NC * NS
BPW = B // NW

CB = 64
NSUB = BPW // CB
NCH = F * NSUB
IDS = CB * L
GW = 128
NG = IDS // GW


def _sc_depad_ids(ids3_hbm, flat_hbm, in0, in1, fl0, fl1,
                  semi, semo0, semo1):
    wid = lax.axis_index("s") * NC + lax.axis_index("c")
    b0 = wid * BPW

    def stage(f, inb):
        pltpu.async_copy(ids3_hbm.at[f, pl.ds(b0, BPW), :], inb, semi)

    def wait_in(inb):
        pltpu.make_async_copy(
            ids3_hbm.at[0, pl.ds(0, BPW), :], inb, semi).wait()

    def flatten(inb, flb):
        def row_body(i, c):
            flb[pl.ds(i * L, LANES)] = inb[i, pl.ds(0, LANES)]
            flb[pl.ds(i * L + L - LANES, LANES)] = inb[i, pl.ds(L - LANES,
                                                                LANES)]
            return c
        lax.fori_loop(0, BPW, row_body, 0, unroll=False)

    def fire_out(f, flb, semo):
        o = f * (B * L) + b0 * L
        pltpu.async_copy(flb, flat_hbm.at[pl.ds(o, BPW * L)], semo)

    def drain_out(flb, semo):
        pltpu.make_async_copy(
            flb, flat_hbm.at[pl.ds(0, BPW * L)], semo).wait()

    def step(f, inb_c, flb, semo, inb_n, do_stage, do_drain):
        wait_in(inb_c)
        if do_stage:
            stage(f + 1, inb_n)
        if do_drain:
            drain_out(flb, semo)
        flatten(inb_c, flb)
        fire_out(f, flb, semo)

    stage(0, in0)
    step(0, in0, fl0, semo0, in1, True, False)
    step(1, in1, fl1, semo1, in0, True, False)

    def pair_body(k, carry):
        f0 = 2 * k
        step(f0, in0, fl0, semo0, in1, True, True)
        step(f0 + 1, in1, fl1, semo1, in0, True, True)
        return carry

    lax.fori_loop(1, F // 2 - 1, pair_body, 0, unroll=False)

    step(F - 2, in0, fl0, semo0, in1, True, True)
    step(F - 1, in1, fl1, semo1, in0, False, True)
    drain_out(fl0, semo0)
    drain_out(fl1, semo1)


def _sc_embedding_bag(ids_hbm, table_hbm, out_hbm,
                      ids0, ids1, rows0, rows1, out0, out1,
                      semg0, semg1, semi, semo0, semo1):
    wid = lax.axis_index("s") * NC + lax.axis_index("c")
    b0 = wid * BPW

    def ids_off(t):
        f = t // NSUB
        bb = b0 + (t % NSUB) * CB
        return f * (B * L) + bb * L, bb, f

    def stage_ids(t, idsb):
        o0, _, _ = ids_off(t)
        pltpu.async_copy(ids_hbm.at[pl.ds(o0, IDS)], idsb, semi)

    def wait_ids(idsb):
        pltpu.make_async_copy(ids_hbm.at[pl.ds(0, IDS)], idsb, semi).wait()

    def fire_gathers(idsb, rowsb, semg):
        for j in range(NG):
            pltpu.async_copy(
                table_hbm.at[idsb.at[pl.ds(j * GW, GW)]],
                rowsb.at[pl.ds(j * GW, GW)],
                semg,
            )

    def drain_gathers(rowsb, semg):
        pltpu.make_async_copy(
            table_hbm.at[pl.ds(0, IDS)], rowsb, semg).wait()

    def compute(rowsb, outb):
        def bag_body(bag, c):
            base = bag * L
            acc0 = rowsb[base, pl.ds(0, LANES)]
            acc1 = rowsb[base, pl.ds(LANES, LANES)]
            for l in range(1, L):
                acc0 = acc0 + rowsb[base + l, pl.ds(0, LANES)]
                acc1 = acc1 + rowsb[base + l, pl.ds(LANES, LANES)]
            outb[bag, pl.ds(0, LANES)] = acc0
            outb[bag, pl.ds(LANES, LANES)] = acc1
            return c

        lax.fori_loop(0, CB, bag_body, 0, unroll=False)

    def fire_out(t, outb, semo):
        _, bb, f = ids_off(t)
        pltpu.async_copy(
            outb, out_hbm.at[pl.ds(bb, CB), pl.ds(f * D, D)], semo)

    def drain_out(outb, semo):
        pltpu.make_async_copy(
            outb, out_hbm.at[pl.ds(0, CB), pl.ds(0, D)], semo).wait()

    pltpu.sync_copy(ids_hbm.at[pl.ds(b0 * L, IDS)], ids0)
    fire_gathers(ids0, rows0, semg0)
    stage_ids(1, ids1)

    def half(t, idsb_n, rowsb_n, semg_n, idsb_c, rowsb_c, semg_c,
             outb, semo, skip_out_drain):
        wait_ids(idsb_n)
        fire_gathers(idsb_n, rowsb_n, semg_n)
        drain_gathers(rowsb_c, semg_c)
        stage_ids(t + 2, idsb_c)
        if not skip_out_drain:
            drain_out(outb, semo)
        compute(rowsb_c, outb)
        fire_out(t, outb, semo)

    def make_pair_body(first_pair):
        def pair_body(k, carry):
            t0 = 2 * k
            half(t0, ids1, rows1, semg1, ids0, rows0, semg0,
                 out0, semo0, first_pair)
            half(t0 + 1, ids0, rows0, semg0, ids1, rows1, semg1,
                 out1, semo1, first_pair)
            return carry
        return pair_body

    make_pair_body(True)(0, 0)
    lax.fori_loop(1, NCH // 2 - 1, make_pair_body(False), 0, unroll=False)

    t = NCH - 2
    wait_ids(ids1)
    fire_gathers(ids1, rows1, semg1)
    drain_gathers(rows0, semg0)
    drain_out(out0, semo0)
    compute(rows0, out0)
    fire_out(t, out0, semo0)

    drain_gathers(rows1, semg1)
    drain_out(out1, semo1)
    compute(rows1, out1)
    fire_out(t + 1, out1, semo1)

    drain_out(out0, semo0)
    drain_out(out1, semo1)


@jax.jit
def _run(indices, table):
    depad = pl.kernel(
        _sc_depad_ids,
        out_type=jax.ShapeDtypeStruct((F * B * L,), jnp.int32),
        mesh=plsc.VectorSubcoreMesh(core_axis_name="c", subcore_axis_name="s"),
        scratch_types=[
            pltpu.VMEM((BPW, L), jnp.int32),
            pltpu.VMEM((BPW, L), jnp.int32),
            pltpu.VMEM((BPW * L,), jnp.int32),
            pltpu.VMEM((BPW * L,), jnp.int32),
            pltpu.SemaphoreType.DMA,
            pltpu.SemaphoreType.DMA,
            pltpu.SemaphoreType.DMA,
        ],
    )
    ids_hbm = depad(indices)
    kern = pl.kernel(
        _sc_embedding_bag,
        out_type=jax.ShapeDtypeStruct((B, F * D), jnp.float32),
        mesh=plsc.VectorSubcoreMesh(core_axis_name="c", subcore_axis_name="s"),
        scratch_types=[
            pltpu.VMEM((IDS,), jnp.int32),
            pltpu.VMEM((IDS,), jnp.int32),
            pltpu.VMEM((IDS, D), jnp.float32),
            pltpu.VMEM((IDS, D), jnp.float32),
            pltpu.VMEM((CB, D), jnp.float32),
            pltpu.VMEM((CB, D), jnp.float32),
            pltpu.SemaphoreType.DMA,
            pltpu.SemaphoreType.DMA,
            pltpu.SemaphoreType.DMA,
            pltpu.SemaphoreType.DMA,
            pltpu.SemaphoreType.DMA,
        ],
        compiler_params=pltpu.CompilerParams(use_tc_tiling_on_sc=False),
    )
    return kern(ids_hbm, table)


def kernel(indices, table):
    return _run(indices.astype(jnp.int32), table.astype(jnp.float32))
